# baseline (speedup 1.0000x reference)
import jax
import jax.numpy as jnp
from jax.experimental import pallas as pl
from jax.experimental.pallas import tpu as pltpu


def _maxpool_kernel(x_ref, o_ref):
    r, _, w = x_ref.shape
    a = jnp.maximum(x_ref[:, 0, :], x_ref[:, 1, :])
    m = jnp.maximum(a, pltpu.roll(a, w - 1, 1))
    lane = jax.lax.broadcasted_iota(jnp.int32, (r, 128), 1)
    g0 = jnp.take_along_axis(m[:, 0:128], (2 * lane) & 127, axis=1)
    g1 = jnp.take_along_axis(m[:, 96:224], (2 * lane + 32) & 127, axis=1)
    out = jnp.where(lane < 64, g0, g1)
    o_ref[...] = out[:, 0 : w // 2]


def kernel(x):
    N, C, H, W = x.shape
    HO, WO = H // 2, W // 2
    NC = N * C
    K = 8
    R = K * HO
    xv = x.reshape(NC * HO, 2, W)
    out = pl.pallas_call(
        _maxpool_kernel,
        grid=(NC // K,),
        in_specs=[pl.BlockSpec((R, 2, W), lambda i: (i, 0, 0))],
        out_specs=pl.BlockSpec((R, WO), lambda i: (i, 0)),
        out_shape=jax.ShapeDtypeStruct((NC * HO, WO), x.dtype),
        compiler_params=pltpu.CompilerParams(
            dimension_semantics=("parallel",),
        ),
    )(xv)
    return out.reshape(N, C, HO, WO)


# speedup vs baseline: 4.4973x; 4.4973x over previous
import jax
import jax.numpy as jnp
from jax.experimental import pallas as pl
from jax.experimental.pallas import tpu as pltpu


def _maxpool_kernel(x_ref, o_ref, a_ref, b_ref):
    k, g, _, w = x_ref.shape
    x = x_ref[...]
    m8 = jnp.maximum(x, pltpu.roll(x, 7, 2))
    a_ref[...] = m8[:, :, :, 0:128]
    b_ref[:, :, :, 0:96] = m8[:, :, :, 128:224]
    ev = pl.Slice(0, 4, 2)
    a = a_ref[:, :, ev, :]
    b = b_ref[:, :, ev, :]
    ma = jnp.maximum(a, pltpu.roll(a, 127, 3))
    mb = jnp.maximum(b, pltpu.roll(b, 127, 3))
    lane = jax.lax.broadcasted_iota(jnp.int32, (k, g, 4, 128), 3)
    idx = (2 * lane) & 127
    ga = jnp.take_along_axis(ma, idx, axis=3)
    gb = jnp.take_along_axis(mb, idx, axis=3)
    out = jnp.where(lane < 64, ga, gb)
    o_ref[...] = out[:, :, :, 0:112]


def kernel(x):
    N, C, H, W = x.shape
    HO, WO = H // 2, W // 2
    NC = N * C
    K = 48
    xv = x.reshape(NC, H // 8, 8, W)
    out = pl.pallas_call(
        _maxpool_kernel,
        grid=(NC // K,),
        in_specs=[pl.BlockSpec((K, H // 8, 8, W), lambda i: (i, 0, 0, 0))],
        out_specs=pl.BlockSpec((K, H // 8, 4, WO), lambda i: (i, 0, 0, 0)),
        out_shape=jax.ShapeDtypeStruct((NC, H // 8, 4, WO), x.dtype),
        scratch_shapes=[
            pltpu.VMEM((K, H // 8, 8, 128), x.dtype),
            pltpu.VMEM((K, H // 8, 8, 128), x.dtype),
        ],
        compiler_params=pltpu.CompilerParams(
            dimension_semantics=("arbitrary",),
        ),
    )(xv)
    return out.reshape(N, C, HO, WO)
